# baseline (speedup 1.0000x reference)
import functools

import jax
import jax.numpy as jnp
from jax import lax
from jax.experimental import pallas as pl
from jax.experimental.pallas import tpu as pltpu
from jax.experimental.pallas import tpu_sc as plsc

B = 8
T = 8192
D = 256
L = 2048
LANES = 16
NC = 2
NS = 16
NW = NC * NS
ROWS_PER_W = (B * L) // NW
SUBS_PER_B = NW // B
CHUNK = 64
NCHUNK = ROWS_PER_W // CHUNK
NBUF = 6
IDS_WIN = ROWS_PER_W + 8
IDS_GROUPS = ROWS_PER_W // LANES

OFF_OFFS = 0
OFF_IDSW = 32
OFF_IDSO = OFF_IDSW + IDS_WIN
OFF_LEN = OFF_IDSO + ROWS_PER_W
IBUF_LEN = OFF_LEN + LANES


@functools.partial(
    pl.kernel,
    out_type=(
        jax.ShapeDtypeStruct((B, L, D), jnp.float32),
        jax.ShapeDtypeStruct((B, L), jnp.int32),
        jax.ShapeDtypeStruct((B,), jnp.int32),
    ),
    mesh=plsc.VectorSubcoreMesh(core_axis_name="c", subcore_axis_name="s"),
    compiler_params=pltpu.CompilerParams(needs_layout_passes=False),
    scratch_types=[
        pltpu.VMEM((IBUF_LEN,), jnp.int32),
        pltpu.VMEM((NCHUNK, CHUNK), jnp.int32),
        pltpu.VMEM((NBUF + 1, CHUNK, D), jnp.float32),
        pltpu.VMEM((ROWS_PER_W,), jnp.int32),
        pltpu.VMEM((LANES,), jnp.int32),
        pltpu.SemaphoreType.DMA((NBUF,)),
        pltpu.SemaphoreType.DMA((NBUF,)),
        pltpu.SemaphoreType.DMA,
    ],
)
def _pad_sc(vals_hbm, ids_hbm, offs_hbm, out_vals, out_ids, out_len,
            ibuf, idx_v, fbuf, ids_out, len_v, gsems, wsems, isem):
    cid = lax.axis_index("c")
    sid = lax.axis_index("s")
    wid = sid * NC + cid
    b = wid // SUBS_PER_B
    l0 = (wid % SUBS_PER_B) * ROWS_PER_W

    pltpu.sync_copy(offs_hbm, ibuf.at[pl.ds(OFF_OFFS, B + 1)])
    lanes = lax.iota(jnp.int32, LANES)
    off_pair = ibuf[pl.ds(OFF_OFFS + b, LANES)]
    off_b = off_pair[0]
    off_b1 = off_pair[1]
    len_b = jnp.minimum(off_b1 - off_b, L)
    nval = jnp.clip(len_b - l0, 0, ROWS_PER_W)

    ws = pl.multiple_of(jnp.minimum((off_b + l0) & ~7, T - IDS_WIN), 8)
    pltpu.async_copy(ids_hbm.at[pl.ds(ws, IDS_WIN)],
                     ibuf.at[pl.ds(OFF_IDSW, IDS_WIN)], isem)

    def cond(c):
        return nval > c * CHUNK

    for g in range(CHUNK // LANES):
        idx_v[0, pl.ds(g * LANES, LANES)] = jnp.minimum(
            off_b + l0 + g * LANES + lanes, T - 1)

    @pl.when(cond(0))
    def _():
        pltpu.async_copy(vals_hbm.at[idx_v.at[0]], fbuf.at[0], gsems.at[0])

    def build_idx(fg, carry):
        c = fg // (CHUNK // LANES)
        g = fg % (CHUNK // LANES)
        base = l0 + fg * LANES
        idx_v[c, pl.ds(g * LANES, LANES)] = jnp.minimum(
            off_b + base + lanes, T - 1)
        return carry
    lax.fori_loop(CHUNK // LANES, NCHUNK * (CHUNK // LANES), build_idx, 0)

    for c in range(1, NBUF):
        @pl.when(cond(c))
        def _(c=c):
            pltpu.async_copy(vals_hbm.at[idx_v.at[c]], fbuf.at[c],
                             gsems.at[c])

    @pl.when(nval < ROWS_PER_W)
    def _():
        zeros16 = jnp.zeros((LANES,), jnp.float32)

        def zrow(r, carry):
            for g in range(D // LANES):
                fbuf[NBUF, r, pl.ds(g * LANES, LANES)] = zeros16
            return carry
        lax.fori_loop(0, CHUNK, zrow, 0)

    pltpu.make_async_copy(ids_hbm.at[pl.ds(ws, IDS_WIN)],
                          ibuf.at[pl.ds(OFF_IDSW, IDS_WIN)], isem).wait()

    def ids_group(g, carry):
        pos = l0 + g * LANES + lanes
        rel = jnp.minimum(off_b + pos, T - 1) - ws
        gathered = plsc.load_gather(ibuf, [rel + OFF_IDSW])
        ids_out[pl.ds(g * LANES, LANES)] = jnp.where(pos < len_b, gathered, 0)
        return carry
    lax.fori_loop(0, IDS_GROUPS, ids_group, 0)
    pltpu.async_copy(ids_out, out_ids.at[b, pl.ds(l0, ROWS_PER_W)], isem)

    @pl.when(wid == 0)
    def _():
        offs = ibuf[pl.ds(OFF_OFFS, LANES)]
        nxt = plsc.load_gather(ibuf, [jnp.minimum(lanes + 1, LANES - 1)])
        len_v[...] = jnp.minimum(nxt - offs, L)
        pltpu.sync_copy(len_v.at[pl.ds(0, B)], out_len)

    for c in range(NCHUNK):
        i = c % NBUF
        cv = nval - c * CHUNK
        dst = out_vals.at[b, pl.ds(l0 + c * CHUNK, CHUNK)]

        @pl.when(cv > 0)
        def _(c=c, i=i, cv=cv, dst=dst):
            pltpu.make_async_copy(vals_hbm.at[idx_v.at[c]], fbuf.at[i],
                                  gsems.at[i]).wait()

            @pl.when(cv < CHUNK)
            def _():
                zeros16 = jnp.zeros((LANES,), jnp.float32)

                def zrow(r, carry):
                    for g in range(D // LANES):
                        fbuf[i, r, pl.ds(g * LANES, LANES)] = zeros16
                    return carry
                lax.fori_loop(cv, CHUNK, zrow, 0)

            pltpu.async_copy(fbuf.at[i], dst, wsems.at[i])

        @pl.when(cv <= 0)
        def _(i=i, dst=dst):
            pltpu.async_copy(fbuf.at[NBUF], dst, wsems.at[i])

        if c + NBUF < NCHUNK:
            @pl.when(cond(c + NBUF))
            def _(c=c, i=i, dst=dst):
                pltpu.make_async_copy(fbuf.at[NBUF], dst, wsems.at[i]).wait()
                pltpu.async_copy(vals_hbm.at[idx_v.at[c + NBUF]], fbuf.at[i],
                                 gsems.at[i])

    for c in range(NCHUNK):
        dst = out_vals.at[b, pl.ds(l0 + c * CHUNK, CHUNK)]
        if c + NBUF >= NCHUNK:
            pltpu.make_async_copy(fbuf.at[NBUF], dst, wsems.at[c % NBUF]).wait()
        else:
            @pl.when(jnp.logical_not(cond(c + NBUF)))
            def _(c=c, dst=dst):
                pltpu.make_async_copy(fbuf.at[NBUF], dst,
                                      wsems.at[c % NBUF]).wait()

    pltpu.make_async_copy(ids_out, out_ids.at[b, pl.ds(l0, ROWS_PER_W)],
                          isem).wait()


def kernel(seq_values, item_ids, offsets):
    return _pad_sc(seq_values, item_ids, offsets)


# speedup vs baseline: 1.1000x; 1.1000x over previous
import functools

import jax
import jax.numpy as jnp
from jax import lax
from jax.experimental import pallas as pl
from jax.experimental.pallas import tpu as pltpu
from jax.experimental.pallas import tpu_sc as plsc

B = 8
T = 8192
D = 256
L = 2048
LANES = 16
NC = 2
NS = 16
NW = NC * NS
ROWS_PER_W = (B * L) // NW
SUBS_PER_B = NW // B
CHUNK = 32
NCHUNK = ROWS_PER_W // CHUNK
NBUF = 8
IDS_WIN = ROWS_PER_W + 8
IDS_GROUPS = ROWS_PER_W // LANES

OFF_OFFS = 0
OFF_IDSW = 32
OFF_IDSO = OFF_IDSW + IDS_WIN
OFF_LEN = OFF_IDSO + ROWS_PER_W
IBUF_LEN = OFF_LEN + LANES


@functools.partial(
    pl.kernel,
    out_type=(
        jax.ShapeDtypeStruct((B, L, D), jnp.float32),
        jax.ShapeDtypeStruct((B, L), jnp.int32),
        jax.ShapeDtypeStruct((B,), jnp.int32),
    ),
    mesh=plsc.VectorSubcoreMesh(core_axis_name="c", subcore_axis_name="s"),
    compiler_params=pltpu.CompilerParams(needs_layout_passes=False),
    scratch_types=[
        pltpu.VMEM((IBUF_LEN,), jnp.int32),
        pltpu.VMEM((NCHUNK, CHUNK), jnp.int32),
        pltpu.VMEM((NBUF + 1, CHUNK, D), jnp.float32),
        pltpu.VMEM((ROWS_PER_W,), jnp.int32),
        pltpu.VMEM((LANES,), jnp.int32),
        pltpu.SemaphoreType.DMA((NBUF,)),
        pltpu.SemaphoreType.DMA((NBUF,)),
        pltpu.SemaphoreType.DMA,
    ],
)
def _pad_sc(vals_hbm, ids_hbm, offs_hbm, out_vals, out_ids, out_len,
            ibuf, idx_v, fbuf, ids_out, len_v, gsems, wsems, isem):
    cid = lax.axis_index("c")
    sid = lax.axis_index("s")
    wid = sid * NC + cid
    b = wid // SUBS_PER_B
    l0 = (wid % SUBS_PER_B) * ROWS_PER_W

    pltpu.sync_copy(offs_hbm, ibuf.at[pl.ds(OFF_OFFS, B + 1)])
    lanes = lax.iota(jnp.int32, LANES)
    off_pair = ibuf[pl.ds(OFF_OFFS + b, LANES)]
    off_b = off_pair[0]
    off_b1 = off_pair[1]
    len_b = jnp.minimum(off_b1 - off_b, L)
    nval = jnp.clip(len_b - l0, 0, ROWS_PER_W)

    ws = pl.multiple_of(jnp.minimum((off_b + l0) & ~7, T - IDS_WIN), 8)
    pltpu.async_copy(ids_hbm.at[pl.ds(ws, IDS_WIN)],
                     ibuf.at[pl.ds(OFF_IDSW, IDS_WIN)], isem)

    def cond(c):
        return nval > c * CHUNK

    for g in range(CHUNK // LANES):
        idx_v[0, pl.ds(g * LANES, LANES)] = jnp.minimum(
            off_b + l0 + g * LANES + lanes, T - 1)

    @pl.when(cond(0))
    def _():
        pltpu.async_copy(vals_hbm.at[idx_v.at[0]], fbuf.at[0], gsems.at[0])

    def build_idx(fg, carry):
        c = fg // (CHUNK // LANES)
        g = fg % (CHUNK // LANES)
        base = l0 + fg * LANES
        idx_v[c, pl.ds(g * LANES, LANES)] = jnp.minimum(
            off_b + base + lanes, T - 1)
        return carry
    lax.fori_loop(CHUNK // LANES, NCHUNK * (CHUNK // LANES), build_idx, 0)

    for c in range(1, NBUF):
        @pl.when(cond(c))
        def _(c=c):
            pltpu.async_copy(vals_hbm.at[idx_v.at[c]], fbuf.at[c],
                             gsems.at[c])

    @pl.when(nval < ROWS_PER_W)
    def _():
        zeros16 = jnp.zeros((LANES,), jnp.float32)

        def zrow(r, carry):
            for g in range(D // LANES):
                fbuf[NBUF, r, pl.ds(g * LANES, LANES)] = zeros16
            return carry
        lax.fori_loop(0, CHUNK, zrow, 0)

    pltpu.make_async_copy(ids_hbm.at[pl.ds(ws, IDS_WIN)],
                          ibuf.at[pl.ds(OFF_IDSW, IDS_WIN)], isem).wait()

    def ids_group(g, carry):
        pos = l0 + g * LANES + lanes
        rel = jnp.minimum(off_b + pos, T - 1) - ws
        gathered = plsc.load_gather(ibuf, [rel + OFF_IDSW])
        ids_out[pl.ds(g * LANES, LANES)] = jnp.where(pos < len_b, gathered, 0)
        return carry
    lax.fori_loop(0, IDS_GROUPS, ids_group, 0)
    pltpu.async_copy(ids_out, out_ids.at[b, pl.ds(l0, ROWS_PER_W)], isem)

    @pl.when(wid == 0)
    def _():
        offs = ibuf[pl.ds(OFF_OFFS, LANES)]
        nxt = plsc.load_gather(ibuf, [jnp.minimum(lanes + 1, LANES - 1)])
        len_v[...] = jnp.minimum(nxt - offs, L)
        pltpu.sync_copy(len_v.at[pl.ds(0, B)], out_len)

    for c in range(NCHUNK):
        i = c % NBUF
        cv = nval - c * CHUNK
        dst = out_vals.at[b, pl.ds(l0 + c * CHUNK, CHUNK)]

        @pl.when(cv > 0)
        def _(c=c, i=i, cv=cv, dst=dst):
            pltpu.make_async_copy(vals_hbm.at[idx_v.at[c]], fbuf.at[i],
                                  gsems.at[i]).wait()

            @pl.when(cv < CHUNK)
            def _():
                zeros16 = jnp.zeros((LANES,), jnp.float32)

                def zrow(r, carry):
                    for g in range(D // LANES):
                        fbuf[i, r, pl.ds(g * LANES, LANES)] = zeros16
                    return carry
                lax.fori_loop(cv, CHUNK, zrow, 0)

            pltpu.async_copy(fbuf.at[i], dst, wsems.at[i])

        @pl.when(cv <= 0)
        def _(i=i, dst=dst):
            pltpu.async_copy(fbuf.at[NBUF], dst, wsems.at[i])

        if c + NBUF < NCHUNK:
            @pl.when(cond(c + NBUF))
            def _(c=c, i=i, dst=dst):
                pltpu.make_async_copy(fbuf.at[NBUF], dst, wsems.at[i]).wait()
                pltpu.async_copy(vals_hbm.at[idx_v.at[c + NBUF]], fbuf.at[i],
                                 gsems.at[i])

    for c in range(NCHUNK):
        dst = out_vals.at[b, pl.ds(l0 + c * CHUNK, CHUNK)]
        if c + NBUF >= NCHUNK:
            pltpu.make_async_copy(fbuf.at[NBUF], dst, wsems.at[c % NBUF]).wait()
        else:
            @pl.when(jnp.logical_not(cond(c + NBUF)))
            def _(c=c, dst=dst):
                pltpu.make_async_copy(fbuf.at[NBUF], dst,
                                      wsems.at[c % NBUF]).wait()

    pltpu.make_async_copy(ids_out, out_ids.at[b, pl.ds(l0, ROWS_PER_W)],
                          isem).wait()


def kernel(seq_values, item_ids, offsets):
    return _pad_sc(seq_values, item_ids, offsets)
